# Initial kernel scaffold; baseline (speedup 1.0000x reference)
import jax
import jax.numpy as jnp
from jax.experimental import pallas as pl


def kernel(idx, embn, embhe, W1, b1, W2, b2):
    raise NotImplementedError("write your pallas kernel here")


# speedup vs baseline: 8.6058x; 8.6058x over previous
import functools

import jax
import jax.numpy as jnp
from jax import lax
from jax.experimental import pallas as pl
from jax.experimental.pallas import tpu as pltpu
from jax.experimental.pallas import tpu_sc as plsc

_ALPHA = 3.0
_K = 20
_B = 4096
_NH = 256
_D = 64
_BLK = 256
_HI_BITS = 0x43800001


@functools.cache
def _make_sc_gather(V, D, B):
  info = plsc.get_sparse_core_info()
  nw = info.num_cores * info.num_subcores
  b_per_w = B // nw
  mesh = plsc.VectorSubcoreMesh(core_axis_name="c", subcore_axis_name="s")

  @functools.partial(
      pl.kernel,
      out_type=jax.ShapeDtypeStruct((B, D), jnp.float32),
      mesh=mesh,
      compiler_params=pltpu.CompilerParams(use_tc_tiling_on_sc=False),
      scratch_types=[
          pltpu.VMEM((b_per_w,), jnp.int32),
          pltpu.VMEM((b_per_w, D), jnp.float32),
          pltpu.SemaphoreType.DMA,
      ],
  )
  def sc_gather(table_hbm, idx_hbm, out_hbm, idx_v, rows_v, sem):
    wid = lax.axis_index("s") * info.num_cores + lax.axis_index("c")
    base = wid * b_per_w
    pltpu.sync_copy(idx_hbm.at[pl.ds(base, b_per_w)], idx_v)
    pltpu.async_copy(table_hbm.at[idx_v], rows_v, sem).wait()
    pltpu.sync_copy(rows_v, out_hbm.at[pl.ds(base, b_per_w)])

  return sc_gather


def _bf16_dot_nt(x, y):
  return lax.dot_general(
      x.astype(jnp.bfloat16), y.astype(jnp.bfloat16),
      (((1,), (1,)), ((), ())), preferred_element_type=jnp.float32)


def _h_body(g_ref, embhe_ref, w1_ref, b1_ref, w2_ref, b2_ref, h_ref):
  nv1 = jnp.tanh(_ALPHA * (_bf16_dot_nt(g_ref[...], w1_ref[...])
                           + b1_ref[...]))
  nv2 = jnp.tanh(_ALPHA * (_bf16_dot_nt(embhe_ref[...], w2_ref[...])
                           + b2_ref[...]))
  h0 = _bf16_dot_nt(nv1, nv2)
  h_ref[...] = jnp.maximum(jnp.tanh(_ALPHA * h0), 0.0).astype(jnp.bfloat16)


def _adj_body(h_blk_ref, h_ref, out_ref):
  a = lax.dot_general(h_blk_ref[...], h_ref[...], (((1,), (1,)), ((), ())),
                      preferred_element_type=jnp.float32)
  ai = lax.bitcast_convert_type(a, jnp.int32)

  lo = jnp.zeros((_BLK, 1), jnp.int32)
  hi = jnp.full((_BLK, 1), _HI_BITS, jnp.int32)

  def body(_, carry):
    lo, hi = carry
    mid = lo + ((hi - lo) >> 1)
    cnt = jnp.sum((ai >= mid).astype(jnp.int32), axis=1, keepdims=True)
    ge = cnt >= _K
    return jnp.where(ge, mid, lo), jnp.where(ge, hi, mid)

  lo, hi = lax.fori_loop(0, 31, body, (lo, hi))

  gt = ai > lo
  eq = ai == lo
  need = _K - jnp.sum(gt.astype(jnp.int32), axis=1, keepdims=True)
  col = lax.broadcasted_iota(jnp.int32, (_BLK, _B), 1)

  def ibody(_, carry):
    clo, chi = carry
    mid = clo + ((chi - clo) >> 1)
    cnt = jnp.sum((eq & (col <= mid)).astype(jnp.int32), axis=1,
                  keepdims=True)
    ge = cnt >= need
    return jnp.where(ge, clo, mid), jnp.where(ge, mid, chi)

  clo, chi = lax.fori_loop(
      0, 12,
      ibody,
      (jnp.full((_BLK, 1), -1, jnp.int32),
       jnp.full((_BLK, 1), _B - 1, jnp.int32)))
  out_ref[...] = jnp.where(gt | (eq & (col <= chi)), a, 0.0)


def _tc_pipeline(g, embhe, W1, b1, W2, b2):
  h = pl.pallas_call(
      _h_body,
      out_shape=jax.ShapeDtypeStruct((_B, _NH), jnp.bfloat16),
  )(g, embhe, W1, b1.reshape(1, _D), W2, b2.reshape(1, _D))

  nblk = _B // _BLK
  adj = pl.pallas_call(
      _adj_body,
      grid=(nblk,),
      in_specs=[
          pl.BlockSpec((_BLK, _NH), lambda i: (i, 0)),
          pl.BlockSpec((_B, _NH), lambda i: (0, 0)),
      ],
      out_specs=pl.BlockSpec((_BLK, _B), lambda i: (i, 0)),
      out_shape=jax.ShapeDtypeStruct((_B, _B), jnp.float32),
  )(h, h)
  return adj


def kernel(idx, embn, embhe, W1, b1, W2, b2):
  idx = idx.astype(jnp.int32)
  g = _make_sc_gather(embn.shape[0], _D, _B)(embn, idx)
  return _tc_pipeline(g, embhe, W1, b1, W2, b2)
